# Initial kernel scaffold; baseline (speedup 1.0000x reference)
import jax
import jax.numpy as jnp
from jax.experimental import pallas as pl


def kernel(x, lifecycle_states, blueprint_ids, grafting_strategies, blend_factors, blueprint_weights):
    raise NotImplementedError("write your pallas kernel here")


# speedup vs baseline: 1.0546x; 1.0546x over previous
import jax
import jax.numpy as jnp
from jax import lax
from jax.experimental import pallas as pl
from jax.experimental.pallas import tpu as pltpu

_S = 64
_CHUNK = 64
_NB = 10


def _body(ls_ref, ids_ref, st_ref, al_ref, bw_ref, x_ref, o_ref, a_ref, c_ref):
    @pl.when(pl.program_id(0) == 0)
    def _compute_coeffs():
        H = x_ref.shape[1]
        ls = ls_ref[...]
        st = st_ref[...]
        al = al_ref[...]
        active = (ls >= 3) & (ls <= 6)
        one = jnp.ones_like(al)
        zero = jnp.zeros_like(al)
        g = jnp.where(active & (st == 0), al,
                      jnp.where(active & (st == 1), one, zero))
        hh = jnp.where(active & (st == 0), one - al,
                       jnp.where(active & (st == 1), zero, one))
        kk = jnp.where(active & (st != 0) & (st != 1), one, zero)
        idsf = ids_ref[...].astype(jnp.float32)
        packed = jnp.concatenate([g, hh, kk, idsf], axis=0)
        row = lax.broadcasted_iota(jnp.int32, (_S, H), 0)
        cols = lax.broadcasted_iota(jnp.int32, (_S, H), 1) // _CHUNK
        E = (row == cols).astype(jnp.float32)
        exp = jnp.dot(packed, E, preferred_element_type=jnp.float32)
        g_col = exp[0:1, :]
        h_col = exp[1:2, :]
        k_col = exp[2:3, :]
        ids_col = exp[3:4, :].astype(jnp.int32)
        jrow = lax.broadcasted_iota(jnp.int32, (_NB, H), 0)
        sel = jnp.where(ids_col == jrow, bw_ref[...], 0.0)
        w_row = jnp.sum(sel, axis=0, keepdims=True)
        a_ref[...] = g_col * w_row + h_col
        c_ref[...] = k_col * w_row

    o_ref[...] = x_ref[...] * a_ref[...] + c_ref[...]


def kernel(x, lifecycle_states, blueprint_ids, grafting_strategies,
           blend_factors, blueprint_weights):
    B, H = x.shape
    R = 256
    grid = (B // R,)
    ls2 = lifecycle_states.reshape(1, _S)
    ids2 = blueprint_ids.reshape(1, _S)
    st2 = grafting_strategies.reshape(1, _S)
    al2 = blend_factors.reshape(1, _S)
    small = lambda: pl.BlockSpec((1, _S), lambda i: (0, 0))
    return pl.pallas_call(
        _body,
        grid=grid,
        in_specs=[
            small(), small(), small(), small(),
            pl.BlockSpec((_NB, H), lambda i: (0, 0)),
            pl.BlockSpec((R, H), lambda i: (i, 0)),
        ],
        out_specs=pl.BlockSpec((R, H), lambda i: (i, 0)),
        out_shape=jax.ShapeDtypeStruct((B, H), x.dtype),
        scratch_shapes=[
            pltpu.VMEM((1, H), jnp.float32),
            pltpu.VMEM((1, H), jnp.float32),
        ],
    )(ls2, ids2, st2, al2, blueprint_weights, x)


# speedup vs baseline: 1.0794x; 1.0235x over previous
import jax
import jax.numpy as jnp
from jax import lax
from jax.experimental import pallas as pl
from jax.experimental.pallas import tpu as pltpu

_S = 64
_CHUNK = 64
_NB = 10


def _body(ls_ref, ids_ref, st_ref, al_ref, bw_ref, x_ref, o_ref, a_ref, c_ref):
    @pl.when(pl.program_id(0) == 0)
    def _compute_coeffs():
        H = x_ref.shape[1]
        ls = ls_ref[...]
        st = st_ref[...]
        al = al_ref[...]
        active = (ls >= 3) & (ls <= 6)
        one = jnp.ones_like(al)
        zero = jnp.zeros_like(al)
        g = jnp.where(active & (st == 0), al,
                      jnp.where(active & (st == 1), one, zero))
        hh = jnp.where(active & (st == 0), one - al,
                       jnp.where(active & (st == 1), zero, one))
        kk = jnp.where(active & (st != 0) & (st != 1), one, zero)
        idsf = ids_ref[...].astype(jnp.float32)
        packed = jnp.concatenate([g, hh, kk, idsf], axis=0)
        row = lax.broadcasted_iota(jnp.int32, (_S, H), 0)
        cols = lax.broadcasted_iota(jnp.int32, (_S, H), 1) // _CHUNK
        E = (row == cols).astype(jnp.float32)
        exp = jnp.dot(packed, E, preferred_element_type=jnp.float32)
        g_col = exp[0:1, :]
        h_col = exp[1:2, :]
        k_col = exp[2:3, :]
        ids_col = exp[3:4, :].astype(jnp.int32)
        jrow = lax.broadcasted_iota(jnp.int32, (_NB, H), 0)
        sel = jnp.where(ids_col == jrow, bw_ref[...], 0.0)
        w_row = jnp.sum(sel, axis=0, keepdims=True)
        a_ref[...] = g_col * w_row + h_col
        c_ref[...] = k_col * w_row

    o_ref[...] = x_ref[...] * a_ref[...] + c_ref[...]


def kernel(x, lifecycle_states, blueprint_ids, grafting_strategies,
           blend_factors, blueprint_weights):
    B, H = x.shape
    R = 512
    grid = (B // R,)
    ls2 = lifecycle_states.reshape(1, _S)
    ids2 = blueprint_ids.reshape(1, _S)
    st2 = grafting_strategies.reshape(1, _S)
    al2 = blend_factors.reshape(1, _S)
    small = lambda: pl.BlockSpec((1, _S), lambda i: (0, 0))
    return pl.pallas_call(
        _body,
        grid=grid,
        in_specs=[
            small(), small(), small(), small(),
            pl.BlockSpec((_NB, H), lambda i: (0, 0)),
            pl.BlockSpec((R, H), lambda i: (i, 0)),
        ],
        out_specs=pl.BlockSpec((R, H), lambda i: (i, 0)),
        out_shape=jax.ShapeDtypeStruct((B, H), x.dtype),
        scratch_shapes=[
            pltpu.VMEM((1, H), jnp.float32),
            pltpu.VMEM((1, H), jnp.float32),
        ],
    )(ls2, ids2, st2, al2, blueprint_weights, x)
